# baseline (speedup 1.0000x reference)
import functools

import jax
import jax.numpy as jnp
from jax import lax
from jax.experimental import pallas as pl
from jax.experimental.pallas import tpu as pltpu
from jax.experimental.pallas import tpu_sc as plsc


BT = 512
F = 1024

NC = 2
NS = 16
NW = NC * NS


def _sc_gather_rows(x, idx_r, bound):
    d = x.shape[1]
    nw, n_chunks, chunk = idx_r.shape
    rows_w = n_chunks * chunk
    n_rows = nw * rows_w
    mesh = plsc.VectorSubcoreMesh(core_axis_name="c", subcore_axis_name="s")

    def body(x_hbm, idx_hbm, bound_hbm, out_hbm, idx_v, bv, buf0, buf1,
             sem0, sem1):
        wid = lax.axis_index("s") * NC + lax.axis_index("c")
        pltpu.sync_copy(bound_hbm, bv)
        pltpu.sync_copy(idx_hbm.at[wid], idx_v)
        base = wid * rows_w
        bufs = (buf0, buf1)
        sems = (sem0, sem1)
        bnd = bv[...][0]
        nv = jnp.clip((bnd - base + chunk - 1) // chunk, 0, n_chunks)

        def make(ch):
            r = ch % 2
            return pltpu.make_async_copy(
                x_hbm.at[idx_v.at[ch]], bufs[r], sems[r])

        descs = [make(ch) for ch in range(n_chunks)]

        @pl.when(nv > 0)
        def _():
            descs[0].start()

        for ch in range(n_chunks):
            def _wait_store(ch=ch):
                descs[ch].wait()

            pl.when(ch < nv)(_wait_store)
            if ch + 1 < n_chunks:
                def _issue(ch=ch):
                    descs[ch + 1].start()

                pl.when(ch + 1 < nv)(_issue)

            def _store(ch=ch):
                pltpu.sync_copy(bufs[ch % 2],
                                out_hbm.at[pl.ds(base + ch * chunk, chunk)])

            pl.when(ch < nv)(_store)

    return pl.kernel(
        body,
        out_type=jax.ShapeDtypeStruct((n_rows, d), x.dtype),
        mesh=mesh,
        scratch_types=[
            pltpu.VMEM((n_chunks, chunk), jnp.int32),
            pltpu.VMEM((16,), jnp.int32),
            pltpu.VMEM((chunk, d), x.dtype),
            pltpu.VMEM((chunk, d), x.dtype),
            pltpu.SemaphoreType.DMA,
            pltpu.SemaphoreType.DMA,
        ],
    )(x, idx_r, bound)


def _sc_combine(ys, dest_r):
    d = ys.shape[1]
    nw, n_chunks, k, chunk = dest_r.shape
    toks_w = n_chunks * chunk
    t = nw * toks_w
    mesh = plsc.VectorSubcoreMesh(core_axis_name="c", subcore_axis_name="s")

    def body(ys_hbm, dest_hbm, out_hbm, idx_v, a0, a1, b0, b1,
             sa0, sa1, sb0, sb1):
        wid = lax.axis_index("s") * NC + lax.axis_index("c")
        pltpu.sync_copy(dest_hbm.at[wid], idx_v)
        base = wid * toks_w
        abufs = (a0, a1)
        bbufs = (b0, b1)
        asems = (sa0, sa1)
        bsems = (sb0, sb1)

        def issue(ch):
            r = ch % 2
            cpa = pltpu.async_copy(ys_hbm.at[idx_v.at[ch, 0]],
                                   abufs[r], asems[r])
            cpb = pltpu.async_copy(ys_hbm.at[idx_v.at[ch, 1]],
                                   bbufs[r], bsems[r])
            return cpa, cpb

        pending = {0: issue(0)}
        for ch in range(n_chunks):
            cpa, cpb = pending[ch]
            cpa.wait()
            cpb.wait()
            if ch + 1 < n_chunks:
                pending[ch + 1] = issue(ch + 1)
            r = ch % 2
            a, b = abufs[r], bbufs[r]

            def col_body(j, _):
                sl = pl.ds(j * 16, 16)

                def row_body(i, _):
                    a[i, sl] = a[i, sl] + b[i, sl]
                    return 0

                return lax.fori_loop(0, chunk, row_body, 0)

            lax.fori_loop(0, d // 16, col_body, 0)
            pltpu.sync_copy(a, out_hbm.at[pl.ds(base + ch * chunk, chunk)])

    return pl.kernel(
        body,
        out_type=jax.ShapeDtypeStruct((t, d), ys.dtype),
        mesh=mesh,
        scratch_types=[
            pltpu.VMEM((n_chunks, k, chunk), jnp.int32),
            pltpu.VMEM((chunk, d), ys.dtype),
            pltpu.VMEM((chunk, d), ys.dtype),
            pltpu.VMEM((chunk, d), ys.dtype),
            pltpu.VMEM((chunk, d), ys.dtype),
            pltpu.SemaphoreType.DMA,
            pltpu.SemaphoreType.DMA,
            pltpu.SemaphoreType.DMA,
            pltpu.SemaphoreType.DMA,
        ],
    )(ys, dest_r)


def _mm_body(be_ref, brow_ref, bval_ref, xs_ref, gate_ref, up_ref, down_ref,
             w_ref, out_ref):
    g = pl.program_id(0)
    j = pl.program_id(1)

    @pl.when(bval_ref[g] == 1)
    def _():
        x_ = xs_ref[...]
        dn = (((1,), (1,)), ((), ()))
        gt = jax.lax.dot_general(x_, gate_ref[0], dn,
                                 preferred_element_type=jnp.float32)
        up = jax.lax.dot_general(x_, up_ref[0], dn,
                                 preferred_element_type=jnp.float32)
        h = gt * jax.nn.sigmoid(gt) * up
        y = jax.lax.dot_general(h, down_ref[0], dn,
                                preferred_element_type=jnp.float32)
        y = y * w_ref[...]

        @pl.when(j == 0)
        def _():
            out_ref[...] = y

        @pl.when(j != 0)
        def _():
            out_ref[...] += y


def _grouped_matmul(xs, gate_up, down, w_pad, block_expert, block_row,
                    block_valid, g_max, nf):
    n_rows, d_model = xs.shape
    ff = gate_up.shape[1] // 2

    grid_spec = pltpu.PrefetchScalarGridSpec(
        num_scalar_prefetch=3,
        grid=(g_max, nf),
        in_specs=[
            pl.BlockSpec((BT, d_model),
                         lambda g, j, be, br, bv: (br[g], 0)),
            pl.BlockSpec((1, F, d_model),
                         lambda g, j, be, br, bv: (be[g], j, 0)),
            pl.BlockSpec((1, F, d_model),
                         lambda g, j, be, br, bv: (be[g], (ff // F) + j, 0)),
            pl.BlockSpec((1, d_model, F),
                         lambda g, j, be, br, bv: (be[g], 0, j)),
            pl.BlockSpec((BT, 1),
                         lambda g, j, be, br, bv: (br[g], 0)),
        ],
        out_specs=pl.BlockSpec((BT, d_model),
                               lambda g, j, be, br, bv: (br[g], 0)),
    )
    return pl.pallas_call(
        _mm_body,
        grid_spec=grid_spec,
        out_shape=jax.ShapeDtypeStruct((n_rows, d_model), jnp.float32),
        compiler_params=pltpu.CompilerParams(
            dimension_semantics=("arbitrary", "arbitrary"),
        ),
    )(block_expert, block_row, block_valid, xs, gate_up, gate_up, down, w_pad)


def kernel(x, gate_up_proj, down_proj, topk_weights, topk_ids):
    t, d_model = x.shape
    e = gate_up_proj.shape[0]
    k = topk_ids.shape[1]
    n = t * k

    g_max = -(-n // BT) + e - 1
    gchunk = 48
    quant = (NW * gchunk) // BT + (1 if (NW * gchunk) % BT else 0)
    g_max = -(-g_max // quant) * quant
    n_rows = g_max * BT
    nf = (gate_up_proj.shape[1] // 2) // F

    flat = topk_ids.reshape(-1).astype(jnp.int32)
    oh = (flat[:, None] == jnp.arange(e, dtype=jnp.int32)[None, :])
    rank_all = jnp.cumsum(oh.astype(jnp.int32), axis=0)
    counts = rank_all[-1]
    rank = jnp.take_along_axis(rank_all, flat[:, None], axis=1)[:, 0] - 1

    blocks_e = -(-counts // BT)
    block_start = jnp.concatenate(
        [jnp.zeros((1,), jnp.int32), jnp.cumsum(blocks_e).astype(jnp.int32)])
    total_blocks = block_start[e]

    g_ids = jnp.arange(g_max, dtype=jnp.int32)
    be = jnp.searchsorted(block_start[1:], g_ids, side="right").astype(jnp.int32)
    be = jnp.minimum(be, e - 1)
    block_valid = (g_ids < total_blocks).astype(jnp.int32)
    last_valid = jnp.maximum(total_blocks - 1, 0)
    block_expert = jnp.where(block_valid == 1, be, be[last_valid])
    block_row = jnp.where(block_valid == 1, g_ids, last_valid)

    pad_off = block_start[:e] * BT
    dest = pad_off[flat] + rank

    tok_of = jnp.arange(n, dtype=jnp.int32) // k
    row_tok = (jnp.arange(n_rows, dtype=jnp.int32) % t).at[dest].set(tok_of)
    w_pad = jnp.zeros((n_rows,), jnp.float32).at[dest].set(
        topk_weights.reshape(-1))

    idx_r = row_tok.reshape(NW, n_rows // (NW * gchunk), gchunk)
    bound = jnp.full((16,), total_blocks * BT, dtype=jnp.int32)
    xs = _sc_gather_rows(x, idx_r, bound)

    ys = _grouped_matmul(xs, gate_up_proj, down_proj,
                         w_pad.reshape(n_rows, 1),
                         block_expert, block_row, block_valid, g_max, nf)

    tchunk = 16
    dest_r = dest.reshape(NW, t // (NW * tchunk), tchunk, k)
    dest_r = dest_r.transpose(0, 1, 3, 2)
    out = _sc_combine(ys, dest_r)
    return out


# speedup vs baseline: 1.0287x; 1.0287x over previous
import functools

import jax
import jax.numpy as jnp
from jax import lax
from jax.experimental import pallas as pl
from jax.experimental.pallas import tpu as pltpu
from jax.experimental.pallas import tpu_sc as plsc


BT = 512
F = 1024

NC = 2
NS = 16
NW = NC * NS


def _sc_gather_rows(x, idx_r, bound):
    d = x.shape[1]
    nw, n_chunks, chunk = idx_r.shape
    rows_w = n_chunks * chunk
    n_rows = nw * rows_w
    mesh = plsc.VectorSubcoreMesh(core_axis_name="c", subcore_axis_name="s")

    def body(x_hbm, idx_hbm, bound_hbm, out_hbm, idx_v, bv, buf0, buf1,
             sem0, sem1):
        wid = lax.axis_index("s") * NC + lax.axis_index("c")
        pltpu.sync_copy(bound_hbm, bv)
        pltpu.sync_copy(idx_hbm.at[wid], idx_v)
        base = wid * rows_w
        bufs = (buf0, buf1)
        sems = (sem0, sem1)
        bnd = bv[...][0]
        nv = jnp.clip((bnd - base + chunk - 1) // chunk, 0, n_chunks)

        def make(ch):
            r = ch % 2
            return pltpu.make_async_copy(
                x_hbm.at[idx_v.at[ch]], bufs[r], sems[r])

        descs = [make(ch) for ch in range(n_chunks)]

        @pl.when(nv > 0)
        def _():
            descs[0].start()

        for ch in range(n_chunks):
            def _wait_store(ch=ch):
                descs[ch].wait()

            pl.when(ch < nv)(_wait_store)
            if ch + 1 < n_chunks:
                def _issue(ch=ch):
                    descs[ch + 1].start()

                pl.when(ch + 1 < nv)(_issue)

            def _store(ch=ch):
                pltpu.sync_copy(bufs[ch % 2],
                                out_hbm.at[pl.ds(base + ch * chunk, chunk)])

            pl.when(ch < nv)(_store)

    return pl.kernel(
        body,
        out_type=jax.ShapeDtypeStruct((n_rows, d), x.dtype),
        mesh=mesh,
        scratch_types=[
            pltpu.VMEM((n_chunks, chunk), jnp.int32),
            pltpu.VMEM((16,), jnp.int32),
            pltpu.VMEM((chunk, d), x.dtype),
            pltpu.VMEM((chunk, d), x.dtype),
            pltpu.SemaphoreType.DMA,
            pltpu.SemaphoreType.DMA,
        ],
    )(x, idx_r, bound)


def _sc_combine(ys, dest_r):
    d = ys.shape[1]
    nw, n_chunks, k, chunk = dest_r.shape
    toks_w = n_chunks * chunk
    t = nw * toks_w
    mesh = plsc.VectorSubcoreMesh(core_axis_name="c", subcore_axis_name="s")

    def body(ys_hbm, dest_hbm, out_hbm, idx_v, a0, a1, b0, b1,
             sa0, sa1, sb0, sb1):
        wid = lax.axis_index("s") * NC + lax.axis_index("c")
        pltpu.sync_copy(dest_hbm.at[wid], idx_v)
        base = wid * toks_w
        abufs = (a0, a1)
        bbufs = (b0, b1)
        asems = (sa0, sa1)
        bsems = (sb0, sb1)

        def issue(ch):
            r = ch % 2
            cpa = pltpu.async_copy(ys_hbm.at[idx_v.at[ch, 0]],
                                   abufs[r], asems[r])
            cpb = pltpu.async_copy(ys_hbm.at[idx_v.at[ch, 1]],
                                   bbufs[r], bsems[r])
            return cpa, cpb

        pending = {0: issue(0)}
        for ch in range(n_chunks):
            cpa, cpb = pending[ch]
            cpa.wait()
            cpb.wait()
            if ch + 1 < n_chunks:
                pending[ch + 1] = issue(ch + 1)
            r = ch % 2
            a, b = abufs[r], bbufs[r]

            def col_body(j, _):
                sl = pl.ds(j * 16, 16)
                for i in range(chunk):
                    a[i, sl] = a[i, sl] + b[i, sl]
                return 0

            lax.fori_loop(0, d // 16, col_body, 0)
            pltpu.sync_copy(a, out_hbm.at[pl.ds(base + ch * chunk, chunk)])

    return pl.kernel(
        body,
        out_type=jax.ShapeDtypeStruct((t, d), ys.dtype),
        mesh=mesh,
        scratch_types=[
            pltpu.VMEM((n_chunks, k, chunk), jnp.int32),
            pltpu.VMEM((chunk, d), ys.dtype),
            pltpu.VMEM((chunk, d), ys.dtype),
            pltpu.VMEM((chunk, d), ys.dtype),
            pltpu.VMEM((chunk, d), ys.dtype),
            pltpu.SemaphoreType.DMA,
            pltpu.SemaphoreType.DMA,
            pltpu.SemaphoreType.DMA,
            pltpu.SemaphoreType.DMA,
        ],
    )(ys, dest_r)


def _mm_body(be_ref, brow_ref, bval_ref, xs_ref, gate_ref, up_ref, down_ref,
             w_ref, out_ref):
    g = pl.program_id(0)
    j = pl.program_id(1)

    @pl.when(bval_ref[g] == 1)
    def _():
        x_ = xs_ref[...]
        dn = (((1,), (1,)), ((), ()))
        gt = jax.lax.dot_general(x_, gate_ref[0], dn,
                                 preferred_element_type=jnp.float32)
        up = jax.lax.dot_general(x_, up_ref[0], dn,
                                 preferred_element_type=jnp.float32)
        h = gt * jax.nn.sigmoid(gt) * up
        y = jax.lax.dot_general(h, down_ref[0], dn,
                                preferred_element_type=jnp.float32)
        y = y * w_ref[...]

        @pl.when(j == 0)
        def _():
            out_ref[...] = y

        @pl.when(j != 0)
        def _():
            out_ref[...] += y


def _grouped_matmul(xs, gate_up, down, w_pad, block_expert, block_row,
                    block_valid, g_max, nf):
    n_rows, d_model = xs.shape
    ff = gate_up.shape[1] // 2

    grid_spec = pltpu.PrefetchScalarGridSpec(
        num_scalar_prefetch=3,
        grid=(g_max, nf),
        in_specs=[
            pl.BlockSpec((BT, d_model),
                         lambda g, j, be, br, bv: (br[g], 0)),
            pl.BlockSpec((1, F, d_model),
                         lambda g, j, be, br, bv: (be[g], j, 0)),
            pl.BlockSpec((1, F, d_model),
                         lambda g, j, be, br, bv: (be[g], (ff // F) + j, 0)),
            pl.BlockSpec((1, d_model, F),
                         lambda g, j, be, br, bv: (be[g], 0, j)),
            pl.BlockSpec((BT, 1),
                         lambda g, j, be, br, bv: (br[g], 0)),
        ],
        out_specs=pl.BlockSpec((BT, d_model),
                               lambda g, j, be, br, bv: (br[g], 0)),
    )
    return pl.pallas_call(
        _mm_body,
        grid_spec=grid_spec,
        out_shape=jax.ShapeDtypeStruct((n_rows, d_model), jnp.float32),
        compiler_params=pltpu.CompilerParams(
            dimension_semantics=("arbitrary", "arbitrary"),
        ),
    )(block_expert, block_row, block_valid, xs, gate_up, gate_up, down, w_pad)


def kernel(x, gate_up_proj, down_proj, topk_weights, topk_ids):
    t, d_model = x.shape
    e = gate_up_proj.shape[0]
    k = topk_ids.shape[1]
    n = t * k

    g_max = -(-n // BT) + e - 1
    gchunk = 32
    quant = (NW * gchunk) // BT + (1 if (NW * gchunk) % BT else 0)
    g_max = -(-g_max // quant) * quant
    n_rows = g_max * BT
    nf = (gate_up_proj.shape[1] // 2) // F

    flat = topk_ids.reshape(-1).astype(jnp.int32)
    oh = (flat[:, None] == jnp.arange(e, dtype=jnp.int32)[None, :])
    rank_all = jnp.cumsum(oh.astype(jnp.int32), axis=0)
    counts = rank_all[-1]
    rank = jnp.take_along_axis(rank_all, flat[:, None], axis=1)[:, 0] - 1

    blocks_e = -(-counts // BT)
    block_start = jnp.concatenate(
        [jnp.zeros((1,), jnp.int32), jnp.cumsum(blocks_e).astype(jnp.int32)])
    total_blocks = block_start[e]

    g_ids = jnp.arange(g_max, dtype=jnp.int32)
    be = jnp.searchsorted(block_start[1:], g_ids, side="right").astype(jnp.int32)
    be = jnp.minimum(be, e - 1)
    block_valid = (g_ids < total_blocks).astype(jnp.int32)
    last_valid = jnp.maximum(total_blocks - 1, 0)
    block_expert = jnp.where(block_valid == 1, be, be[last_valid])
    block_row = jnp.where(block_valid == 1, g_ids, last_valid)

    pad_off = block_start[:e] * BT
    dest = pad_off[flat] + rank

    tok_of = jnp.arange(n, dtype=jnp.int32) // k
    row_tok = (jnp.arange(n_rows, dtype=jnp.int32) % t).at[dest].set(tok_of)
    w_pad = jnp.zeros((n_rows,), jnp.float32).at[dest].set(
        topk_weights.reshape(-1))

    idx_r = row_tok.reshape(NW, n_rows // (NW * gchunk), gchunk)
    bound = jnp.full((16,), total_blocks * BT, dtype=jnp.int32)
    xs = _sc_gather_rows(x, idx_r, bound)

    ys = _grouped_matmul(xs, gate_up_proj, down_proj,
                         w_pad.reshape(n_rows, 1),
                         block_expert, block_row, block_valid, g_max, nf)

    tchunk = 16
    dest_r = dest.reshape(NW, t // (NW * tchunk), tchunk, k)
    dest_r = dest_r.transpose(0, 1, 3, 2)
    out = _sc_combine(ys, dest_r)
    return out
